# Initial kernel scaffold; baseline (speedup 1.0000x reference)
import jax
import jax.numpy as jnp
from jax.experimental import pallas as pl


def kernel(x, edge_index, W1, lin_W, lin_b, nodes, target_label, steps):
    raise NotImplementedError("write your pallas kernel here")


# speedup vs baseline: 2.6801x; 2.6801x over previous
import jax
import jax.numpy as jnp
from jax import lax
from jax.experimental import pallas as pl
from jax.experimental.pallas import tpu as pltpu
from jax.experimental.pallas import tpu_sc as plsc

N = 10000
NP = 10240
E = 160000
D = 64
DP = 128
NT = 16
EPT_R = E // NT
EPT = 10240
EP = EPT * NT
CH = 64
NCH = EPT // CH
NPT = NP // NT
F32 = jnp.float32
I32 = jnp.int32


def _mm_body(x_ref, w_ref, o_ref):
    o_ref[...] = jnp.dot(x_ref[...], w_ref[...], preferred_element_type=F32)


def _project(x, w1):
    return pl.pallas_call(
        _mm_body,
        grid=(10,),
        in_specs=[
            pl.BlockSpec((N // 10, 256), lambda i: (i, 0)),
            pl.BlockSpec((256, D), lambda i: (0, 0)),
        ],
        out_specs=pl.BlockSpec((N // 10, D), lambda i: (i, 0)),
        out_shape=jax.ShapeDtypeStruct((N, D), F32),
    )(x, w1)


def _rsqrt16(x):
    i = lax.bitcast_convert_type(x, I32)
    i = jnp.full((16,), 0x5F3759DF, I32) - lax.shift_right_logical(i, 1)
    y = lax.bitcast_convert_type(i, F32)
    for _ in range(4):
        y = y * (1.5 - 0.5 * x * y * y)
    return y


def _splat(v, dtype=I32):
    return jnp.full((16,), v, dtype)


_BCAST_DN = lax.GatherDimensionNumbers(
    offset_dims=(), collapsed_slice_dims=(0,), start_index_map=(0,))


def _bcast(v16, lane):
    idx = jnp.full((16,), lane, I32)
    return lax.gather(v16, idx[:, None], _BCAST_DN, (1,),
                      mode=lax.GatherScatterMode.PROMISE_IN_BOUNDS)


def _lanesum(v16):
    iota = lax.iota(I32, 16)
    for sh in (8, 4, 2, 1):
        idx = jnp.bitwise_xor(iota, sh)
        v16 = v16 + lax.gather(v16, idx[:, None], _BCAST_DN, (1,),
                               mode=lax.GatherScatterMode.PROMISE_IN_BOUNDS)
    return v16


def _ka_body(m_hbm, srcw_hbm, dstw_hbm, h_hbm, w_hbm,
             u_hbm, dinv_hbm,
             m_v, hr_v, msg_v, au_v, uu_v, nb_v,
             ds_v, dd_v, si_v, di_v, idn_v, w_v, ps16,
             deg_sh, dinv_sh, agg_sh, psum_sh):
    cid = lax.axis_index("c")

    @pl.when(cid == 0)
    def _core0():
        tid = lax.axis_index("s")
        ebase = tid * EPT
        nbase = tid * NPT
        iota = lax.iota(I32, 16)
        zeros16 = jnp.zeros((16,), F32)

        pltpu.sync_copy(m_hbm.at[pl.ds(ebase, EPT)], m_v)
        pltpu.sync_copy(w_hbm, w_v)

        def _zero_nb(j, c):
            nb_v[pl.ds(j * 16, 16)] = zeros16
            return c
        lax.fori_loop(0, NPT // 16, _zero_nb, 0)
        pltpu.sync_copy(nb_v, deg_sh.at[pl.ds(nbase, NPT)])

        def _zero_msg(r, c):
            def seg2(s, c2):
                msg_v[r, pl.ds(s * 16, 16)] = zeros16
                return c2
            lax.fori_loop(0, D // 16, seg2, 0)
            return c
        lax.fori_loop(0, CH, _zero_msg, 0)

        def _zero_agg(cb, c):
            pltpu.sync_copy(msg_v, agg_sh.at[pl.ds(nbase + cb * CH, CH)])
            return c
        lax.fori_loop(0, NPT // CH, _zero_agg, 0)
        plsc.subcore_barrier()

        def _deg(ch, c):
            pltpu.sync_copy(dstw_hbm.at[tid * NCH + ch], di_v)
            pltpu.sync_copy(m_v.at[pl.ds(ch * CH, CH)],
                            deg_sh.at[di_v], add=True)
            return c
        lax.fori_loop(0, NCH, _deg, 0)
        plsc.subcore_barrier()

        pltpu.sync_copy(deg_sh.at[pl.ds(nbase, NPT)], nb_v)

        def _dinv(j, c):
            s = pl.ds(j * 16, 16)
            nb_v[s] = _rsqrt16(nb_v[s] + 1e-8)
            return c
        lax.fori_loop(0, NPT // 16, _dinv, 0)
        pltpu.sync_copy(nb_v, dinv_sh.at[pl.ds(nbase, NPT)])
        pltpu.sync_copy(nb_v, dinv_hbm.at[pl.ds(nbase, NPT)])
        plsc.subcore_barrier()

        def _agg(ch, c):
            pltpu.sync_copy(srcw_hbm.at[tid * NCH + ch], si_v)
            pltpu.sync_copy(dstw_hbm.at[tid * NCH + ch], di_v)
            pltpu.sync_copy(h_hbm.at[si_v], hr_v)
            pltpu.sync_copy(dinv_sh.at[si_v], ds_v)
            pltpu.sync_copy(dinv_sh.at[di_v], dd_v)

            def _grp(j16, c2):
                sj = pl.ds(j16 * 16, 16)
                s = pl.ds(ch * CH + j16 * 16, 16)
                n16 = ds_v[sj] * m_v[s] * dd_v[sj]

                def _lane(l, c3):
                    nb = _bcast(n16, l)
                    r = j16 * 16 + l

                    def _seg(sg, c4):
                        sl = pl.ds(sg * 16, 16)
                        msg_v[r, sl] = nb * hr_v[r, sl]
                        return c4
                    lax.fori_loop(0, D // 16, _seg, 0)
                    return c3
                lax.fori_loop(0, 16, _lane, 0)
                return c2
            lax.fori_loop(0, CH // 16, _grp, 0)
            pltpu.sync_copy(msg_v, agg_sh.at[di_v], add=True)
            return c
        lax.fori_loop(0, NCH, _agg, 0)
        plsc.subcore_barrier()

        def _u(cb, c):
            def _idn(j, c2):
                idn_v[pl.ds(j * 16, 16)] = iota + _splat(
                    nbase + cb * CH + j * 16)
                return c2
            lax.fori_loop(0, CH // 16, _idn, 0)
            pltpu.sync_copy(agg_sh.at[idn_v], au_v)

            def _row(r, c2):
                def _seg(sg, c3):
                    sl = pl.ds(sg * 16, 16)
                    a = au_v[r, sl]
                    uu_v[r, sl] = jnp.where(a > 0, w_v[sl], 0.0)
                    return c3
                lax.fori_loop(0, D // 16, _seg, 0)

                def _segz(sg, c3):
                    uu_v[r, pl.ds(D + sg * 16, 16)] = zeros16
                    return c3
                lax.fori_loop(0, (DP - D) // 16, _segz, 0)
                return c2
            lax.fori_loop(0, CH, _row, 0)
            pltpu.sync_copy(uu_v, u_hbm.at[pl.ds(nbase + cb * CH, CH)])
            return c
        lax.fori_loop(0, NPT // CH, _u, 0)


def _kb_body(m_hbm, srcw_hbm, dstw_hbm, h_hbm, u_in, dinv_in, par_hbm,
             out_hbm,
             m_v, cbuf, hr_v, ur_v, nb_v, dslice_v,
             ds_v, dd_v, gb_v, pd_v, ps_v, si_v, di_v, par_v, ps16,
             psall_v,
             dinv_sh, r_sh, t_sh, psum_sh):
    cid = lax.axis_index("c")

    @pl.when(cid == 0)
    def _core0():
        tid = lax.axis_index("s")
        ebase = tid * EPT
        nbase = tid * NPT
        iota = lax.iota(I32, 16)
        zeros16 = jnp.zeros((16,), F32)

        pltpu.sync_copy(m_hbm.at[pl.ds(ebase, EPT)], m_v)
        pltpu.sync_copy(par_hbm, par_v)
        pltpu.sync_copy(dinv_in.at[pl.ds(nbase, NPT)], nb_v)
        pltpu.sync_copy(nb_v, dinv_sh.at[pl.ds(nbase, NPT)])

        def _cp_d(j, c):
            s = pl.ds(j * 16, 16)
            dslice_v[s] = nb_v[s]
            return c
        lax.fori_loop(0, NPT // 16, _cp_d, 0)

        def _zero_nb(j, c):
            nb_v[pl.ds(j * 16, 16)] = zeros16
            return c
        lax.fori_loop(0, NPT // 16, _zero_nb, 0)
        pltpu.sync_copy(nb_v, r_sh.at[pl.ds(nbase, NPT)])

        acc = lax.fori_loop(
            0, EPT // 16,
            lambda j, a: a + m_v[pl.ds(j * 16, 16)],
            jnp.zeros((16,), F32))
        ps16[...] = acc
        pltpu.sync_copy(ps16, psum_sh.at[pl.ds(tid * 16, 16)])
        plsc.subcore_barrier()

        pltpu.sync_copy(psum_sh, psall_v)
        acc2 = lax.fori_loop(
            0, NT,
            lambda j, a: a + psall_v[pl.ds(j * 16, 16)],
            jnp.zeros((16,), F32))
        ssum = _lanesum(acc2)

        def _cphase(ch, c):
            pltpu.sync_copy(srcw_hbm.at[tid * NCH + ch], si_v)
            pltpu.sync_copy(dstw_hbm.at[tid * NCH + ch], di_v)
            pltpu.sync_copy(h_hbm.at[si_v], hr_v)
            pltpu.sync_copy(u_in.at[di_v], ur_v)
            pltpu.sync_copy(dinv_sh.at[si_v], ds_v)
            pltpu.sync_copy(dinv_sh.at[di_v], dd_v)

            def _rows(j16, c2):
                def _edge(l, cacc):
                    r = j16 * 16 + l
                    part = jnp.zeros((16,), F32)
                    for sg in range(D // 16):
                        sl = pl.ds(sg * 16, 16)
                        part = part + hr_v[r, sl] * ur_v[r, sl]
                    dall = _lanesum(part)
                    return jnp.where(iota == l, dall, cacc)
                c16 = lax.fori_loop(0, 16, _edge, jnp.zeros((16,), F32))
                s = pl.ds(ch * CH + j16 * 16, 16)
                cbuf[s] = c16
                p16 = c16 * m_v[s]
                sj = pl.ds(j16 * 16, 16)
                pd_v[sj] = p16 * dd_v[sj]
                ps_v[sj] = p16 * ds_v[sj]
                return c2
            lax.fori_loop(0, CH // 16, _rows, 0)
            pltpu.sync_copy(pd_v, r_sh.at[si_v], add=True)
            pltpu.sync_copy(ps_v, r_sh.at[di_v], add=True)
            return c
        lax.fori_loop(0, NCH, _cphase, 0)
        plsc.subcore_barrier()

        pltpu.sync_copy(r_sh.at[pl.ds(nbase, NPT)], nb_v)

        def _t(j, c):
            s = pl.ds(j * 16, 16)
            d16 = dslice_v[s]
            nb_v[s] = -0.5 * d16 * d16 * d16 * nb_v[s]
            return c
        lax.fori_loop(0, NPT // 16, _t, 0)
        pltpu.sync_copy(nb_v, t_sh.at[pl.ds(nbase, NPT)])
        plsc.subcore_barrier()

        anc = par_v[pl.ds(0, 16)]
        gb16 = 2.0 * (anc - ssum)

        def _upd(ch, c):
            pltpu.sync_copy(srcw_hbm.at[tid * NCH + ch], si_v)
            pltpu.sync_copy(dstw_hbm.at[tid * NCH + ch], di_v)
            pltpu.sync_copy(t_sh.at[di_v], gb_v)
            pltpu.sync_copy(dinv_sh.at[si_v], ds_v)
            pltpu.sync_copy(dinv_sh.at[di_v], dd_v)

            def _grp(j16, c2):
                sj = pl.ds(j16 * 16, 16)
                s = pl.ds(ch * CH + j16 * 16, 16)
                g16 = (cbuf[s] * ds_v[sj] * dd_v[sj]
                       + gb_v[sj] + gb16)
                m_v[s] = jnp.clip(m_v[s] + 0.005 * g16, 0.0, 1.0)
                return c2
            lax.fori_loop(0, CH // 16, _grp, 0)
            return c
        lax.fori_loop(0, NCH, _upd, 0)

        def _padz(j, c):
            m_v[pl.ds(j * 16, 16)] = zeros16
            return c
        lax.fori_loop(EPT_R // 16, EPT // 16, _padz, 0)
        pltpu.sync_copy(m_v, out_hbm.at[pl.ds(ebase, EPT)])


_SC_MESH = plsc.VectorSubcoreMesh(core_axis_name="c", subcore_axis_name="s")

_sc_a = pl.kernel(
    _ka_body,
    out_type=(jax.ShapeDtypeStruct((NP, DP), F32),
              jax.ShapeDtypeStruct((NP,), F32)),
    mesh=_SC_MESH,
    scratch_types=[
        pltpu.VMEM((EPT,), F32),
        pltpu.VMEM((CH, DP), F32),
        pltpu.VMEM((CH, D), F32),
        pltpu.VMEM((CH, D), F32),
        pltpu.VMEM((CH, DP), F32),
        pltpu.VMEM((NPT,), F32),
        pltpu.VMEM((CH,), F32),
        pltpu.VMEM((CH,), F32),
        pltpu.VMEM((CH,), I32),
        pltpu.VMEM((CH,), I32),
        pltpu.VMEM((CH,), I32),
        pltpu.VMEM((D,), F32),
        pltpu.VMEM((16,), F32),
        pltpu.VMEM_SHARED((NP,), F32),
        pltpu.VMEM_SHARED((NP,), F32),
        pltpu.VMEM_SHARED((NP, D), F32),
        pltpu.VMEM_SHARED((256,), F32),
    ],
)

_sc_b = pl.kernel(
    _kb_body,
    out_type=jax.ShapeDtypeStruct((EP,), F32),
    mesh=_SC_MESH,
    scratch_types=[
        pltpu.VMEM((EPT,), F32),
        pltpu.VMEM((EPT,), F32),
        pltpu.VMEM((CH, DP), F32),
        pltpu.VMEM((CH, DP), F32),
        pltpu.VMEM((NPT,), F32),
        pltpu.VMEM((NPT,), F32),
        pltpu.VMEM((CH,), F32),
        pltpu.VMEM((CH,), F32),
        pltpu.VMEM((CH,), F32),
        pltpu.VMEM((CH,), F32),
        pltpu.VMEM((CH,), F32),
        pltpu.VMEM((CH,), I32),
        pltpu.VMEM((CH,), I32),
        pltpu.VMEM((16,), F32),
        pltpu.VMEM((16,), F32),
        pltpu.VMEM((256,), F32),
        pltpu.VMEM_SHARED((NP,), F32),
        pltpu.VMEM_SHARED((NP,), F32),
        pltpu.VMEM_SHARED((NP,), F32),
        pltpu.VMEM_SHARED((256,), F32),
    ],
)


def kernel(x, edge_index, W1, lin_W, lin_b, nodes, target_label, steps):
    h = _project(x, W1)
    h_p = jnp.pad(h, ((0, NP - N), (0, DP - D)))
    label = jnp.asarray(target_label)
    wN = jnp.take(lin_W, label, axis=1).astype(F32) / N
    anchor = jnp.where(label == 0, 9.0, 8.0).astype(F32)
    par = jnp.full((16,), anchor, F32)

    src = edge_index[0].astype(I32).reshape(NT, EPT_R)
    dst = edge_index[1].astype(I32).reshape(NT, EPT_R)
    pad = ((0, 0), (0, EPT - EPT_R))
    src_p = jnp.pad(src, pad, constant_values=NP - 1)
    dst_p = jnp.pad(dst, pad, constant_values=NP - 1)
    srcw = src_p.reshape(NT * NCH, CH)
    dstw = dst_p.reshape(NT * NCH, CH)
    m0 = jnp.pad(jnp.full((NT, EPT_R), 0.5, F32), pad).reshape(-1)

    def _step(_, m):
        u, dinv = _sc_a(m, srcw, dstw, h_p, wN)
        return _sc_b(m, srcw, dstw, h_p, u, dinv, par)

    m_fin = lax.fori_loop(0, steps, _step, m0)
    return m_fin.reshape(NT, EPT)[:, :EPT_R].reshape(-1)
